# baseline (speedup 1.0000x reference)
import functools
import math

import jax
import jax.numpy as jnp
from jax import lax
from jax.experimental import pallas as pl
from jax.experimental.pallas import tpu as pltpu
from jax.experimental.pallas import tpu_sc as plsc

B, S, M, D, L = 1024, 50, 24, 64, 4
NW = 32
BW = B // NW
NCHUNK = S
ROWS = M * BW
MPL = M // L


def _time_embed_body(td_ref, mask_ref, divf_ref, phase_ref, out_ref):
    td = td_ref[...] * mask_ref[...]
    row = lax.broadcasted_iota(jnp.int32, (S, S), 0)
    col = lax.broadcasted_iota(jnp.int32, (S, S), 1)
    tri = (col < row).astype(jnp.float32)
    t = jnp.dot(tri, td, preferred_element_type=jnp.float32,
                precision=lax.Precision.HIGHEST)
    arg = t[:, :, None] * divf_ref[...][0][None, None, :] + phase_ref[...][0][None, None, :]
    out_ref[...] = jnp.sin(arg)


def _time_embed(td_t, mask_t, divf, phase):
    bb = 256
    return pl.pallas_call(
        _time_embed_body,
        grid=(B // bb,),
        in_specs=[
            pl.BlockSpec((S, bb), lambda i: (0, i)),
            pl.BlockSpec((S, bb), lambda i: (0, i)),
            pl.BlockSpec((1, D), lambda i: (0, 0)),
            pl.BlockSpec((1, D), lambda i: (0, 0)),
        ],
        out_specs=pl.BlockSpec((S, bb, D), lambda i: (0, i, 0)),
        out_shape=jax.ShapeDtypeStruct((S, B, D), jnp.float32),
    )(td_t, mask_t, divf, phase)


def _sc_body(idx_hbm, te_hbm, table_hbm, out_hbm,
             idx0, idx1, rows0, rows1, te0, te1, out0, out1,
             si0, si1, sg0, sg1, so0, so1):
    wid = lax.axis_index("s") * 2 + lax.axis_index("c")
    b0 = wid * BW

    idx = (idx0, idx1)
    rows = (rows0, rows1)
    te = (te0, te1)
    out = (out0, out1)
    si = (si0, si1)
    sg = (sg0, sg1)
    so = (so0, so1)

    def fire_idx(c, b):
        pltpu.async_copy(idx_hbm.at[c, :, pl.ds(b0, BW)], idx[b], si[b])

    def stage(c, b):
        pltpu.make_async_copy(
            idx_hbm.at[c, :, pl.ds(b0, BW)], idx[b], si[b]).wait()
        for m in range(M):
            pltpu.async_copy(table_hbm.at[idx[b].at[m]],
                             rows[b].at[pl.ds(m * BW, BW)], sg[b])
        pltpu.async_copy(te_hbm.at[c, pl.ds(b0, BW), :], te[b], sg[b])

    def wait_stage(c, b):
        for m in range(M):
            pltpu.make_async_copy(table_hbm.at[idx[b].at[m]],
                                  rows[b].at[pl.ds(m * BW, BW)], sg[b]).wait()
        pltpu.make_async_copy(te_hbm.at[c, pl.ds(b0, BW), :], te[b], sg[b]).wait()

    lane = lax.iota(jnp.int32, 16)

    def consume(c, b, fire_next, first_out):
        wait_stage(c, b)
        if fire_next:
            fire_idx(c + 2, b)
        if not first_out:
            pltpu.make_async_copy(
                out[b], out_hbm.at[c, :, :, pl.ds(b0, BW)], so[b]).wait()
        rv, tv, ov = rows[b], te[b], out[b]

        @plsc.parallel_loop(0, BW, unroll=2)
        def batch_body(p):
            for db in range(D // 16):
                sl = pl.ds(db * 16, 16)
                acc = tv[p, sl]
                d_idx = db * 16 + lane
                for lev in range(L):
                    for j in range(MPL):
                        acc = acc + rv[(lev * MPL + j) * BW + p, sl]
                    plsc.store_scatter(
                        ov,
                        [jnp.full((16,), lev, jnp.int32), d_idx,
                         jnp.full((16,), 1, jnp.int32) * p],
                        acc)

        pltpu.async_copy(out[b], out_hbm.at[c, :, :, pl.ds(b0, BW)], so[b])

    def wait_out(c, b):
        pltpu.make_async_copy(
            out[b], out_hbm.at[c, :, :, pl.ds(b0, BW)], so[b]).wait()

    fire_idx(0, 0)
    fire_idx(1, 1)
    stage(0, 0)
    stage(1, 1)
    consume(0, 0, True, True)
    stage(2, 0)
    consume(1, 1, True, True)
    stage(3, 1)

    def loop_body(k, carry):
        c0 = 2 * k
        consume(c0, 0, True, False)
        stage(c0 + 2, 0)
        consume(c0 + 1, 1, True, False)
        stage(c0 + 3, 1)
        return carry

    lax.fori_loop(1, NCHUNK // 2 - 1, loop_body, 0)

    consume(NCHUNK - 2, 0, False, False)
    consume(NCHUNK - 1, 1, False, False)
    wait_out(NCHUNK - 2, 0)
    wait_out(NCHUNK - 1, 1)


@functools.partial(
    pl.kernel,
    out_type=jax.ShapeDtypeStruct((S, L, D, B), jnp.float32),
    mesh=plsc.VectorSubcoreMesh(core_axis_name="c", subcore_axis_name="s"),
    compiler_params=pltpu.CompilerParams(use_tc_tiling_on_sc=False,
                                         needs_layout_passes=False),
    scratch_types=[
        pltpu.VMEM((M, BW), jnp.int32),
        pltpu.VMEM((M, BW), jnp.int32),
        pltpu.VMEM((ROWS, D), jnp.float32),
        pltpu.VMEM((ROWS, D), jnp.float32),
        pltpu.VMEM((BW, D), jnp.float32),
        pltpu.VMEM((BW, D), jnp.float32),
        pltpu.VMEM((L, D, BW), jnp.float32),
        pltpu.VMEM((L, D, BW), jnp.float32),
        pltpu.SemaphoreType.DMA,
        pltpu.SemaphoreType.DMA,
        pltpu.SemaphoreType.DMA,
        pltpu.SemaphoreType.DMA,
        pltpu.SemaphoreType.DMA,
        pltpu.SemaphoreType.DMA,
    ],
)
def _sc_gather(idx_hbm, te_hbm, table_hbm, out_hbm,
               idx0, idx1, rows0, rows1, te0, te1, out0, out1,
               si0, si1, sg0, sg1, so0, so1):
    _sc_body(idx_hbm, te_hbm, table_hbm, out_hbm,
             idx0, idx1, rows0, rows1, te0, te1, out0, out1,
             si0, si1, sg0, sg1, so0, so1)


def kernel(dynamic_indices, time_delta, event_mask, table, sin_div_term, cos_div_term):
    idx_t = dynamic_indices.astype(jnp.int32).transpose(1, 2, 0)
    td_t = time_delta.T
    mask_t = event_mask.astype(jnp.float32).T
    divf = jnp.stack([sin_div_term, cos_div_term], axis=-1).reshape(1, D)
    phase = jnp.tile(jnp.array([0.0, math.pi / 2], jnp.float32), D // 2).reshape(1, D)
    te = _time_embed(td_t, mask_t, divf, phase)
    out_t = _sc_gather(idx_t, te, table)
    return out_t.transpose(3, 0, 1, 2)
